# Initial kernel scaffold; baseline (speedup 1.0000x reference)
import jax
import jax.numpy as jnp
from jax.experimental import pallas as pl


def kernel(src_embedding, tgt_embedding, src, tgt, temperature, is_corr):
    raise NotImplementedError("write your pallas kernel here")


# speedup vs baseline: 2.3206x; 2.3206x over previous
import math

import jax
import jax.numpy as jnp
import numpy as np
from jax.experimental import pallas as pl

_TI = 256

_PAIRS = [(0, 1), (0, 2), (0, 3), (1, 2), (1, 3), (2, 3)]
_N_SWEEPS = 8


def _mask44(pred):
    r = jax.lax.broadcasted_iota(jnp.int32, (4, 4), 0)
    c = jax.lax.broadcasted_iota(jnp.int32, (4, 4), 1)
    return pred(r, c).astype(jnp.float32)


def _stats_kernel(se_ref, te_ref, src_ref, tgt_ref, temp_ref,
                  w_ref, c_ref, sw_ref, sxc_ref, syr_ref, syc_ref, araw_ref):
    ti = _TI
    n = te_ref.shape[2]
    se = se_ref[0]
    te = te_ref[0]
    s = jax.lax.dot_general(se, te, (((0,), (0,)), ((), ())),
                            preferred_element_type=jnp.float32)
    sc = temp_ref[0, 0, 0] * (s / math.sqrt(se_ref.shape[1]))
    m = jnp.max(sc, axis=1, keepdims=True)
    iota = jax.lax.broadcasted_iota(jnp.int32, (ti, n), 1)
    cor = jnp.min(jnp.where(sc == m, iota, n), axis=1, keepdims=True)
    p = jnp.exp(sc - m)
    ssum = jnp.sum(p, axis=1, keepdims=True)
    w = 1.0 / ssum
    w_ref[0, 0] = w
    c_ref[0, 0] = cor

    x = src_ref[0]
    tg = tgt_ref[0]
    ohw = jnp.where(iota == cor, w, 0.0)
    colw = jnp.sum(ohw, axis=0, keepdims=True)
    pmat = jax.lax.dot_general(x, ohw, (((1,), (0,)), ((), ())),
                               preferred_element_type=jnp.float32)
    araw_p = jax.lax.dot_general(pmat, tg, (((1,), (1,)), ((), ())),
                                 preferred_element_type=jnp.float32)
    sw_p = jnp.sum(w, axis=0, keepdims=True)
    sxc_p = jax.lax.dot_general(x, w, (((1,), (0,)), ((), ())),
                                preferred_element_type=jnp.float32)
    syr_p = jax.lax.dot_general(colw, tg, (((1,), (1,)), ((), ())),
                                preferred_element_type=jnp.float32)
    syc_p = jax.lax.dot_general(tg, colw, (((1,), (1,)), ((), ())),
                                preferred_element_type=jnp.float32)

    first = pl.program_id(1) == 0

    @pl.when(first)
    def _init():
        sw_ref[0] = sw_p
        sxc_ref[0] = sxc_p
        syr_ref[0] = syr_p
        syc_ref[0] = syc_p
        araw_ref[0] = araw_p

    @pl.when(jnp.logical_not(first))
    def _acc():
        sw_ref[0] += sw_p
        sxc_ref[0] += sxc_p
        syr_ref[0] += syr_p
        syc_ref[0] += syc_p
        araw_ref[0] += araw_p


def _bmm4(a, b):
    out = a[:, :, 0:1] * b[:, 0:1, :]
    for k in range(1, 4):
        out = out + a[:, :, k:k + 1] * b[:, k:k + 1, :]
    return out


def _solve_kernel(sw_ref, sxc_ref, syr_ref, syc_ref, araw_ref, r_ref, t_ref):
    eps = 1e-07
    sw = sw_ref[...]
    den = sw + eps
    c = sw / den
    mux_c = sxc_ref[...] / den
    muy_r = syr_ref[...] / den
    muy_c = syc_ref[...] / den
    a = araw_ref[...] / den - (2.0 - c) * (mux_c * muy_r)

    def e(i, j):
        return a[:, i:i + 1, j:j + 1]

    s00, s01, s02 = e(0, 0), e(0, 1), e(0, 2)
    s10, s11, s12 = e(1, 0), e(1, 1), e(1, 2)
    s20, s21, s22 = e(2, 0), e(2, 1), e(2, 2)
    row0 = jnp.concatenate([s00 + s11 + s22, s12 - s21, s20 - s02, s01 - s10], 2)
    row1 = jnp.concatenate([s12 - s21, s00 - s11 - s22, s01 + s10, s20 + s02], 2)
    row2 = jnp.concatenate([s20 - s02, s01 + s10, -s00 + s11 - s22, s12 + s21], 2)
    row3 = jnp.concatenate([s01 - s10, s20 + s02, s12 + s21, -s00 - s11 + s22], 2)
    kmat = jnp.concatenate([row0, row1, row2, row3], 1)

    bsz = kmat.shape[0]
    i4 = _mask44(lambda r, c: r == c)
    vmat = jnp.broadcast_to(i4[None], (bsz, 4, 4))
    for _ in range(_N_SWEEPS):
        for (p, q) in _PAIRS:
            apq = kmat[:, p:p + 1, q:q + 1]
            diff = kmat[:, q:q + 1, q:q + 1] - kmat[:, p:p + 1, p:p + 1]
            small = jnp.abs(apq) < 1e-30
            theta = diff / jnp.where(small, 1.0, 2.0 * apq)
            t = jnp.sign(theta) / (jnp.abs(theta) + jnp.sqrt(theta * theta + 1.0))
            cc = 1.0 / jnp.sqrt(t * t + 1.0)
            ss = t * cc
            cc = jnp.where(small, 1.0, cc)
            ss = jnp.where(small, 0.0, ss)
            epp_qq = _mask44(
                lambda r, c: ((r == p) & (c == p)) | ((r == q) & (c == q)))
            epq_qp = (_mask44(lambda r, c: (r == p) & (c == q))
                      - _mask44(lambda r, c: (r == q) & (c == p)))
            g = i4[None] + (cc - 1.0) * epp_qq[None] + ss * epq_qp[None]
            gt = i4[None] + (cc - 1.0) * epp_qq[None] - ss * epq_qp[None]
            kmat = _bmm4(_bmm4(gt, kmat), g)
            vmat = _bmm4(vmat, g)

    dvec = jnp.sum(kmat * i4[None], axis=2, keepdims=True)
    dmax = jnp.max(dvec, axis=1, keepdims=True)
    iot = jax.lax.broadcasted_iota(jnp.int32, (bsz, 4, 1), 1)
    jsel = jnp.min(jnp.where(dvec == dmax, iot, 4), axis=1, keepdims=True)
    ohsel = (iot == jsel).astype(jnp.float32)
    qv = vmat[:, :, 0:1] * ohsel[:, 0:1, :]
    for cidx in range(1, 4):
        qv = qv + vmat[:, :, cidx:cidx + 1] * ohsel[:, cidx:cidx + 1, :]
    qv = qv / jnp.sqrt(jnp.sum(qv * qv, axis=1, keepdims=True))
    q0 = qv[:, 0:1, :]
    qx = qv[:, 1:2, :]
    qy = qv[:, 2:3, :]
    qz = qv[:, 3:4, :]
    r0 = jnp.concatenate([q0 * q0 + qx * qx - qy * qy - qz * qz,
                          2.0 * (qx * qy - q0 * qz),
                          2.0 * (qx * qz + q0 * qy)], 2)
    r1 = jnp.concatenate([2.0 * (qy * qx + q0 * qz),
                          q0 * q0 - qx * qx + qy * qy - qz * qz,
                          2.0 * (qy * qz - q0 * qx)], 2)
    r2 = jnp.concatenate([2.0 * (qz * qx - q0 * qy),
                          2.0 * (qz * qy + q0 * qx),
                          q0 * q0 - qx * qx - qy * qy + qz * qz], 2)
    rmat = jnp.concatenate([r0, r1, r2], 1)
    r_ref[...] = rmat

    rmux = rmat[:, :, 0:1] * mux_c[:, 0:1, :]
    for bidx in range(1, 3):
        rmux = rmux + rmat[:, :, bidx:bidx + 1] * mux_c[:, bidx:bidx + 1, :]
    t_ref[...] = muy_c - rmux


def kernel(src_embedding, tgt_embedding, src, tgt, temperature, is_corr):
    B, d, N = src_embedding.shape
    ti = _TI
    ni = N // ti
    temp3 = temperature.reshape(B, 1, 1)

    out_shape = [
        jax.ShapeDtypeStruct((B, ni, ti, 1), jnp.float32),
        jax.ShapeDtypeStruct((B, ni, ti, 1), jnp.int32),
        jax.ShapeDtypeStruct((B, 1, 1), jnp.float32),
        jax.ShapeDtypeStruct((B, 3, 1), jnp.float32),
        jax.ShapeDtypeStruct((B, 1, 3), jnp.float32),
        jax.ShapeDtypeStruct((B, 3, 1), jnp.float32),
        jax.ShapeDtypeStruct((B, 3, 3), jnp.float32),
    ]
    in_specs = [
        pl.BlockSpec((1, d, ti), lambda b, i: (b, 0, i)),
        pl.BlockSpec((1, d, N), lambda b, i: (b, 0, 0)),
        pl.BlockSpec((1, 3, ti), lambda b, i: (b, 0, i)),
        pl.BlockSpec((1, 3, N), lambda b, i: (b, 0, 0)),
        pl.BlockSpec((1, 1, 1), lambda b, i: (b, 0, 0)),
    ]
    out_specs = [
        pl.BlockSpec((1, 1, ti, 1), lambda b, i: (b, i, 0, 0)),
        pl.BlockSpec((1, 1, ti, 1), lambda b, i: (b, i, 0, 0)),
        pl.BlockSpec((1, 1, 1), lambda b, i: (b, 0, 0)),
        pl.BlockSpec((1, 3, 1), lambda b, i: (b, 0, 0)),
        pl.BlockSpec((1, 1, 3), lambda b, i: (b, 0, 0)),
        pl.BlockSpec((1, 3, 1), lambda b, i: (b, 0, 0)),
        pl.BlockSpec((1, 3, 3), lambda b, i: (b, 0, 0)),
    ]
    weight4, corres4, sw, sxc, syr, syc, araw = pl.pallas_call(
        _stats_kernel,
        grid=(B, ni),
        in_specs=in_specs,
        out_specs=out_specs,
        out_shape=out_shape,
    )(src_embedding, tgt_embedding, src, tgt, temp3)

    rmat, tvec = pl.pallas_call(
        _solve_kernel,
        out_shape=[
            jax.ShapeDtypeStruct((B, 3, 3), jnp.float32),
            jax.ShapeDtypeStruct((B, 3, 1), jnp.float32),
        ],
    )(sw, sxc, syr, syc, araw)

    return (rmat, tvec.reshape(B, 3),
            corres4.reshape(B, N, 1), weight4.reshape(B, N, 1))
